# baseline (speedup 1.0000x reference)
import functools

import jax
import jax.numpy as jnp
from jax import lax
from jax.experimental import pallas as pl
from jax.experimental.pallas import tpu as pltpu
from jax.experimental.pallas import tpu_sc as plsc

B = 16384
F_CAT = 26
F_NUM = 13
V = 100000
D = 8
H = 128

NC = 2
NS = 16
NW = NC * NS

VP = 100096
GV = 1024
NGF = 98
NFULL = F_CAT * NGF
T_PAIRS = (NFULL + 2 * NW - 1) // (2 * NW)

TOTAL = B * F_CAT
RPW = TOTAL // NW
CH = 128
NCH = RPW // CH


def _transpose_sc(emb_t, tail_lin):
    mesh = plsc.VectorSubcoreMesh(core_axis_name="c", subcore_axis_name="s")

    @functools.partial(
        pl.kernel,
        mesh=mesh,
        compiler_params=pltpu.CompilerParams(
            use_tc_tiling_on_sc=True, needs_layout_passes=False
        ),
        out_type=jax.ShapeDtypeStruct((F_CAT * VP * D,), jnp.float32),
        scratch_types=[
            pltpu.VMEM((D, GV), jnp.float32),
            pltpu.VMEM((D, GV), jnp.float32),
            pltpu.VMEM((D * GV,), jnp.float32),
            pltpu.VMEM((D * GV,), jnp.float32),
            pltpu.VMEM((D * 32,), jnp.float32),
            pltpu.SemaphoreType.DMA,
            pltpu.SemaphoreType.DMA,
            pltpu.SemaphoreType.DMA,
        ],
    )
    def k(tab_hbm, tail_hbm, out_hbm, in_a, in_b, ob_a, ob_b, ob_p, sI, sOa, sOb):
        wid = lax.axis_index("s") * NC + lax.axis_index("c")
        iota = lax.iota(jnp.int32, 16)
        r_idx = lax.rem(iota, D)
        v_pat = lax.div(iota, D)

        def group_of(t):
            gid = wid + NW * t
            live = jnp.logical_and(gid >= 0, gid < NFULL)
            gc = jnp.clip(gid, 0, NFULL - 1)
            f = lax.div(gc, NGF)
            g = lax.rem(gc, NGF)
            voff = jnp.where(g == NGF - 1, 98944, g * GV)
            return live, f, voff

        def issue_in(t, buf):
            live, f, voff = group_of(t)

            @pl.when(live)
            def _():
                pltpu.async_copy(tab_hbm.at[f, :, pl.ds(voff, GV)], buf, sI)

        def transpose_group(in_buf, out_buf):
            @plsc.parallel_loop(0, GV // 32, unroll=8)
            def body(c):
                for m in range(16):
                    q = c * 16 + m
                    c_idx = v_pat + 2 * q
                    vals = plsc.load_gather(in_buf, [r_idx, c_idx])
                    out_buf[pl.ds(16 * q, 16)] = vals

        def issue_out(t, buf, sem):
            live, f, voff = group_of(t)

            @pl.when(live)
            def _():
                pltpu.async_copy(
                    buf, out_hbm.at[pl.ds((f * VP + voff) * D, D * GV)], sem
                )

        def wait_in(t, buf):
            live, f, voff = group_of(t)

            @pl.when(live)
            def _():
                pltpu.make_async_copy(
                    tab_hbm.at[f, :, pl.ds(voff, GV)], buf, sI
                ).wait()

        def wait_out(t, buf, sem):
            live, f, voff = group_of(t)

            @pl.when(live)
            def _():
                pltpu.make_async_copy(
                    buf, out_hbm.at[pl.ds((f * VP + voff) * D, D * GV)], sem
                ).wait()

        def phase(t_cur, in_cur, in_nxt, ob_cur, sem):
            wait_in(t_cur, in_cur)
            issue_in(t_cur + 1, in_nxt)
            wait_out(t_cur - 2, ob_cur, sem)
            live, _, _ = group_of(t_cur)

            @pl.when(live)
            def _():
                transpose_group(in_cur, ob_cur)

            issue_out(t_cur, ob_cur, sem)

        issue_in(0, in_a)

        def pair(tt, carry):
            phase(2 * tt, in_a, in_b, ob_a, sOa)
            phase(2 * tt + 1, in_b, in_a, ob_b, sOb)
            return carry

        lax.fori_loop(0, T_PAIRS, pair, 0)
        wait_out(2 * T_PAIRS - 2, ob_a, sOa)
        wait_out(2 * T_PAIRS - 1, ob_b, sOb)

        @pl.when(wid < F_CAT)
        def _():
            f = wid
            pltpu.sync_copy(tail_hbm.at[pl.ds(f * (D * 32), D * 32)], ob_p)
            pltpu.sync_copy(ob_p, out_hbm.at[pl.ds((f * VP + 99968) * D, D * 32)])

    return k(emb_t, tail_lin)


def _gather_sc(table, idx):
    mesh = plsc.VectorSubcoreMesh(core_axis_name="c", subcore_axis_name="s")

    @functools.partial(
        pl.kernel,
        mesh=mesh,
        compiler_params=pltpu.CompilerParams(use_tc_tiling_on_sc=False),
        out_type=jax.ShapeDtypeStruct((NW, NCH, CH, D), jnp.float32),
        scratch_types=[
            pltpu.VMEM((NCH, CH), jnp.int32),
            pltpu.VMEM((NCH, CH, D), jnp.float32),
            pltpu.SemaphoreType.DMA,
        ],
    )
    def k(table_hbm, idx_hbm, out_hbm, idx_v, rows_v, sem):
        wid = lax.axis_index("s") * NC + lax.axis_index("c")
        pltpu.sync_copy(idx_hbm.at[wid], idx_v)

        def fire(j, carry):
            pltpu.async_copy(table_hbm.at[idx_v.at[j]], rows_v.at[j], sem)
            return carry

        lax.fori_loop(0, NCH, fire, 0)

        def drain(j, carry):
            pltpu.make_async_copy(
                table_hbm.at[idx_v.at[j]], rows_v.at[j], sem
            ).wait()
            return carry

        lax.fori_loop(0, NCH, drain, 0)
        pltpu.sync_copy(rows_v, out_hbm.at[wid])

    return k(table, idx)


def _mlp_body(x_ref, nv_ref, w1c_ref, w1n_ref, b1_ref, w2_ref, b2_ref, o_ref):
    h = jnp.dot(x_ref[...], w1c_ref[...], preferred_element_type=jnp.float32)
    h = h + jnp.dot(nv_ref[...], w1n_ref[...], preferred_element_type=jnp.float32)
    h = jnp.maximum(h + b1_ref[...], 0.0)
    o = jnp.sum(h * w2_ref[...], axis=1, keepdims=True) + b2_ref[0, 0]
    o_ref[...] = jax.nn.sigmoid(o)


def _mlp_tc(cat_flat, num_values, w1c, w1n, b1f, w2r, b2s):
    blk = 1024
    return pl.pallas_call(
        _mlp_body,
        grid=(B // blk,),
        in_specs=[
            pl.BlockSpec((blk, F_CAT * D), lambda i: (i, 0)),
            pl.BlockSpec((blk, F_NUM), lambda i: (i, 0)),
            pl.BlockSpec((F_CAT * D, H), lambda i: (0, 0)),
            pl.BlockSpec((F_NUM, H), lambda i: (0, 0)),
            pl.BlockSpec((1, H), lambda i: (0, 0)),
            pl.BlockSpec((1, H), lambda i: (0, 0)),
            pl.BlockSpec((1, 1), lambda i: (0, 0)),
        ],
        out_specs=pl.BlockSpec((blk, 1), lambda i: (i, 0)),
        out_shape=jax.ShapeDtypeStruct((B, 1), jnp.float32),
    )(cat_flat, num_values, w1c, w1n, b1f, w2r, b2s)


def kernel(cat_indices, num_values, emb_tables, num_W, num_b, W1, b1, W2, b2):
    emb_t = emb_tables.transpose(0, 2, 1)
    tail_lin = emb_tables[:, 99968:, :].reshape(-1)
    table1d = _transpose_sc(emb_t, tail_lin)
    table = table1d.reshape(F_CAT * VP, D)
    offs = (jnp.arange(F_CAT, dtype=jnp.int32) * VP)[None, :]
    idx = (cat_indices.astype(jnp.int32) + offs).reshape(NW, NCH, CH)
    rows = _gather_sc(table, idx)
    cat_flat = rows.reshape(B, F_CAT * D)

    w1c = W1[: F_CAT * D]
    w1r = W1[F_CAT * D :]
    w1n = jnp.einsum("fd,fdh->fh", num_W, w1r.reshape(F_NUM, D, H))
    b1f = (b1 + num_b.reshape(1, F_NUM * D) @ w1r).reshape(1, H)
    w2r = W2.reshape(1, H)
    b2s = b2.reshape(1, 1)
    return _mlp_tc(cat_flat, num_values, w1c, w1n, b1f, w2r, b2s)


# speedup vs baseline: 1.4120x; 1.4120x over previous
import functools

import jax
import jax.numpy as jnp
from jax import lax
from jax.experimental import pallas as pl
from jax.experimental.pallas import tpu as pltpu
from jax.experimental.pallas import tpu_sc as plsc

B = 16384
F_CAT = 26
F_NUM = 13
V = 100000
D = 8
H = 128

NC = 2
NS = 16
NW = NC * NS

VP = 100096
GV = 1024
NGF = 98
NFULL = F_CAT * NGF
T_PAIRS = (NFULL + 2 * NW - 1) // (2 * NW)

TOTAL = B * F_CAT
RPW = TOTAL // NW
CH = 128
NCH = RPW // CH


def _transpose_sc(emb_t, tail_lin):
    mesh = plsc.VectorSubcoreMesh(core_axis_name="c", subcore_axis_name="s")

    @functools.partial(
        pl.kernel,
        mesh=mesh,
        compiler_params=pltpu.CompilerParams(
            use_tc_tiling_on_sc=True, needs_layout_passes=False
        ),
        out_type=jax.ShapeDtypeStruct((F_CAT * VP * D,), jnp.float32),
        scratch_types=[
            pltpu.VMEM((D, GV), jnp.float32),
            pltpu.VMEM((D, GV), jnp.float32),
            pltpu.VMEM((D * GV,), jnp.float32),
            pltpu.VMEM((D * GV,), jnp.float32),
            pltpu.VMEM((D * 32,), jnp.float32),
            pltpu.SemaphoreType.DMA,
            pltpu.SemaphoreType.DMA,
            pltpu.SemaphoreType.DMA,
        ],
    )
    def k(tab_hbm, tail_hbm, out_hbm, in_a, in_b, ob_a, ob_b, ob_p, sI, sOa, sOb):
        wid = lax.axis_index("s") * NC + lax.axis_index("c")
        iota = lax.iota(jnp.int32, 16)
        r_idx = lax.rem(iota, D)
        v_pat = lax.div(iota, D)

        def group_of(t):
            gid = wid + NW * t
            live = jnp.logical_and(gid >= 0, gid < NFULL)
            gc = jnp.clip(gid, 0, NFULL - 1)
            f = lax.div(gc, NGF)
            g = lax.rem(gc, NGF)
            voff = jnp.where(g == NGF - 1, 98944, g * GV)
            return live, f, voff

        def issue_in(t, buf):
            live, f, voff = group_of(t)

            @pl.when(live)
            def _():
                pltpu.async_copy(tab_hbm.at[f, :, pl.ds(voff, GV)], buf, sI)

        def transpose_group(in_buf, out_buf):
            @plsc.parallel_loop(0, GV // 16, unroll=4)
            def body(c):
                for m in range(8):
                    q = c * 8 + m
                    c_idx = v_pat + 2 * q
                    vals = plsc.load_gather(in_buf, [r_idx, c_idx])
                    out_buf[pl.ds(16 * q, 16)] = vals

        def issue_out(t, buf, sem):
            live, f, voff = group_of(t)

            @pl.when(live)
            def _():
                pltpu.async_copy(
                    buf, out_hbm.at[pl.ds((f * VP + voff) * D, D * GV)], sem
                )

        def wait_in(t, buf):
            live, f, voff = group_of(t)

            @pl.when(live)
            def _():
                pltpu.make_async_copy(
                    tab_hbm.at[f, :, pl.ds(voff, GV)], buf, sI
                ).wait()

        def wait_out(t, buf, sem):
            live, f, voff = group_of(t)

            @pl.when(live)
            def _():
                pltpu.make_async_copy(
                    buf, out_hbm.at[pl.ds((f * VP + voff) * D, D * GV)], sem
                ).wait()

        def phase(t_cur, in_cur, in_nxt, ob_cur, sem):
            wait_in(t_cur, in_cur)
            issue_in(t_cur + 1, in_nxt)
            wait_out(t_cur - 2, ob_cur, sem)
            live, _, _ = group_of(t_cur)

            @pl.when(live)
            def _():
                transpose_group(in_cur, ob_cur)

            issue_out(t_cur, ob_cur, sem)

        issue_in(0, in_a)

        def pair(tt, carry):
            phase(2 * tt, in_a, in_b, ob_a, sOa)
            phase(2 * tt + 1, in_b, in_a, ob_b, sOb)
            return carry

        lax.fori_loop(0, T_PAIRS, pair, 0)
        wait_out(2 * T_PAIRS - 2, ob_a, sOa)
        wait_out(2 * T_PAIRS - 1, ob_b, sOb)

        @pl.when(wid < F_CAT)
        def _():
            f = wid
            pltpu.sync_copy(tail_hbm.at[pl.ds(f * (D * 32), D * 32)], ob_p)
            pltpu.sync_copy(ob_p, out_hbm.at[pl.ds((f * VP + 99968) * D, D * 32)])

    return k(emb_t, tail_lin)


def _gather_sc(table, idx):
    mesh = plsc.VectorSubcoreMesh(core_axis_name="c", subcore_axis_name="s")

    @functools.partial(
        pl.kernel,
        mesh=mesh,
        compiler_params=pltpu.CompilerParams(use_tc_tiling_on_sc=False),
        out_type=jax.ShapeDtypeStruct((NW, NCH, CH, D), jnp.float32),
        scratch_types=[
            pltpu.VMEM((NCH, CH), jnp.int32),
            pltpu.VMEM((NCH, CH, D), jnp.float32),
            pltpu.SemaphoreType.DMA,
        ],
    )
    def k(table_hbm, idx_hbm, out_hbm, idx_v, rows_v, sem):
        wid = lax.axis_index("s") * NC + lax.axis_index("c")
        pltpu.sync_copy(idx_hbm.at[wid], idx_v)

        def fire(j, carry):
            pltpu.async_copy(table_hbm.at[idx_v.at[j]], rows_v.at[j], sem)
            return carry

        lax.fori_loop(0, NCH, fire, 0)

        def drain(j, carry):
            pltpu.make_async_copy(
                table_hbm.at[idx_v.at[j]], rows_v.at[j], sem
            ).wait()
            return carry

        lax.fori_loop(0, NCH, drain, 0)
        pltpu.sync_copy(rows_v, out_hbm.at[wid])

    return k(table, idx)


def _mlp_body(x_ref, nv_ref, w1c_ref, w1n_ref, b1_ref, w2_ref, b2_ref, o_ref):
    h = jnp.dot(x_ref[...], w1c_ref[...], preferred_element_type=jnp.float32)
    h = h + jnp.dot(nv_ref[...], w1n_ref[...], preferred_element_type=jnp.float32)
    h = jnp.maximum(h + b1_ref[...], 0.0)
    o = jnp.sum(h * w2_ref[...], axis=1, keepdims=True) + b2_ref[0, 0]
    o_ref[...] = jax.nn.sigmoid(o)


def _mlp_tc(cat_flat, num_values, w1c, w1n, b1f, w2r, b2s):
    blk = 1024
    return pl.pallas_call(
        _mlp_body,
        grid=(B // blk,),
        in_specs=[
            pl.BlockSpec((blk, F_CAT * D), lambda i: (i, 0)),
            pl.BlockSpec((blk, F_NUM), lambda i: (i, 0)),
            pl.BlockSpec((F_CAT * D, H), lambda i: (0, 0)),
            pl.BlockSpec((F_NUM, H), lambda i: (0, 0)),
            pl.BlockSpec((1, H), lambda i: (0, 0)),
            pl.BlockSpec((1, H), lambda i: (0, 0)),
            pl.BlockSpec((1, 1), lambda i: (0, 0)),
        ],
        out_specs=pl.BlockSpec((blk, 1), lambda i: (i, 0)),
        out_shape=jax.ShapeDtypeStruct((B, 1), jnp.float32),
    )(cat_flat, num_values, w1c, w1n, b1f, w2r, b2s)


def kernel(cat_indices, num_values, emb_tables, num_W, num_b, W1, b1, W2, b2):
    emb_t = emb_tables.transpose(0, 2, 1)
    tail_lin = emb_tables[:, 99968:, :].reshape(-1)
    table1d = _transpose_sc(emb_t, tail_lin)
    table = table1d.reshape(F_CAT * VP, D)
    offs = (jnp.arange(F_CAT, dtype=jnp.int32) * VP)[None, :]
    idx = (cat_indices.astype(jnp.int32) + offs).reshape(NW, NCH, CH)
    rows = _gather_sc(table, idx)
    cat_flat = rows.reshape(B, F_CAT * D)

    w1c = W1[: F_CAT * D]
    w1r = W1[F_CAT * D :]
    w1n = jnp.einsum("fd,fdh->fh", num_W, w1r.reshape(F_NUM, D, H))
    b1f = (b1 + num_b.reshape(1, F_NUM * D) @ w1r).reshape(1, H)
    w2r = W2.reshape(1, H)
    b2s = b2.reshape(1, 1)
    return _mlp_tc(cat_flat, num_values, w1c, w1n, b1f, w2r, b2s)


# speedup vs baseline: 1.4297x; 1.0126x over previous
import functools

import jax
import jax.numpy as jnp
from jax import lax
from jax.experimental import pallas as pl
from jax.experimental.pallas import tpu as pltpu
from jax.experimental.pallas import tpu_sc as plsc

B = 16384
F_CAT = 26
F_NUM = 13
V = 100000
D = 8
H = 128

NC = 2
NS = 16
NW = NC * NS

VP = 100096
GV = 1024
NGF = 98
NFULL = F_CAT * NGF
T_PAIRS = (NFULL + 2 * NW - 1) // (2 * NW)

TOTAL = B * F_CAT
RPW = TOTAL // NW
CH = 128
NCH = RPW // CH


def _transpose_sc(emb_t, tail_lin):
    mesh = plsc.VectorSubcoreMesh(core_axis_name="c", subcore_axis_name="s")

    @functools.partial(
        pl.kernel,
        mesh=mesh,
        compiler_params=pltpu.CompilerParams(
            use_tc_tiling_on_sc=True, needs_layout_passes=False
        ),
        out_type=jax.ShapeDtypeStruct((F_CAT * VP * D,), jnp.float32),
        scratch_types=[
            pltpu.VMEM((D, GV), jnp.float32),
            pltpu.VMEM((D, GV), jnp.float32),
            pltpu.VMEM((D * GV,), jnp.float32),
            pltpu.VMEM((D * GV,), jnp.float32),
            pltpu.VMEM((D * 32,), jnp.float32),
            pltpu.SemaphoreType.DMA,
            pltpu.SemaphoreType.DMA,
            pltpu.SemaphoreType.DMA,
        ],
    )
    def k(tab_hbm, tail_hbm, out_hbm, in_a, in_b, ob_a, ob_b, ob_p, sI, sOa, sOb):
        wid = lax.axis_index("s") * NC + lax.axis_index("c")
        iota = lax.iota(jnp.int32, 16)
        r_idx = lax.rem(iota, D)
        v_pat = lax.div(iota, D)

        def group_of(t):
            gid = wid + NW * t
            live = jnp.logical_and(gid >= 0, gid < NFULL)
            gc = jnp.clip(gid, 0, NFULL - 1)
            f = lax.div(gc, NGF)
            g = lax.rem(gc, NGF)
            voff = jnp.where(g == NGF - 1, 98944, g * GV)
            return live, f, voff

        def issue_in(t, buf):
            live, f, voff = group_of(t)

            @pl.when(live)
            def _():
                pltpu.async_copy(tab_hbm.at[f, :, pl.ds(voff, GV)], buf, sI)

        def transpose_group(in_buf, out_buf):
            @plsc.parallel_loop(0, GV // 8, unroll=4)
            def body(c):
                for m in range(4):
                    q = c * 4 + m
                    c_idx = v_pat + 2 * q
                    vals = plsc.load_gather(in_buf, [r_idx, c_idx])
                    out_buf[pl.ds(16 * q, 16)] = vals

        def issue_out(t, buf, sem):
            live, f, voff = group_of(t)

            @pl.when(live)
            def _():
                pltpu.async_copy(
                    buf, out_hbm.at[pl.ds((f * VP + voff) * D, D * GV)], sem
                )

        def wait_in(t, buf):
            live, f, voff = group_of(t)

            @pl.when(live)
            def _():
                pltpu.make_async_copy(
                    tab_hbm.at[f, :, pl.ds(voff, GV)], buf, sI
                ).wait()

        def wait_out(t, buf, sem):
            live, f, voff = group_of(t)

            @pl.when(live)
            def _():
                pltpu.make_async_copy(
                    buf, out_hbm.at[pl.ds((f * VP + voff) * D, D * GV)], sem
                ).wait()

        def phase(t_cur, in_cur, in_nxt, ob_cur, sem):
            wait_in(t_cur, in_cur)
            issue_in(t_cur + 1, in_nxt)
            wait_out(t_cur - 2, ob_cur, sem)
            live, _, _ = group_of(t_cur)

            @pl.when(live)
            def _():
                transpose_group(in_cur, ob_cur)

            issue_out(t_cur, ob_cur, sem)

        issue_in(0, in_a)

        def pair(tt, carry):
            phase(2 * tt, in_a, in_b, ob_a, sOa)
            phase(2 * tt + 1, in_b, in_a, ob_b, sOb)
            return carry

        lax.fori_loop(0, T_PAIRS, pair, 0)
        wait_out(2 * T_PAIRS - 2, ob_a, sOa)
        wait_out(2 * T_PAIRS - 1, ob_b, sOb)

        @pl.when(wid < F_CAT)
        def _():
            f = wid
            pltpu.sync_copy(tail_hbm.at[pl.ds(f * (D * 32), D * 32)], ob_p)
            pltpu.sync_copy(ob_p, out_hbm.at[pl.ds((f * VP + 99968) * D, D * 32)])

    return k(emb_t, tail_lin)


def _gather_sc(table, idx):
    mesh = plsc.VectorSubcoreMesh(core_axis_name="c", subcore_axis_name="s")

    @functools.partial(
        pl.kernel,
        mesh=mesh,
        compiler_params=pltpu.CompilerParams(use_tc_tiling_on_sc=False),
        out_type=jax.ShapeDtypeStruct((NW, NCH, CH, D), jnp.float32),
        scratch_types=[
            pltpu.VMEM((NCH, CH), jnp.int32),
            pltpu.VMEM((NCH, CH, D), jnp.float32),
            pltpu.SemaphoreType.DMA,
        ],
    )
    def k(table_hbm, idx_hbm, out_hbm, idx_v, rows_v, sem):
        wid = lax.axis_index("s") * NC + lax.axis_index("c")
        pltpu.sync_copy(idx_hbm.at[wid], idx_v)

        def fire(j, carry):
            pltpu.async_copy(table_hbm.at[idx_v.at[j]], rows_v.at[j], sem)
            return carry

        lax.fori_loop(0, NCH, fire, 0)

        def drain(j, carry):
            pltpu.make_async_copy(
                table_hbm.at[idx_v.at[j]], rows_v.at[j], sem
            ).wait()
            return carry

        lax.fori_loop(0, NCH, drain, 0)
        pltpu.sync_copy(rows_v, out_hbm.at[wid])

    return k(table, idx)


def _mlp_body(x_ref, nv_ref, w1c_ref, w1n_ref, b1_ref, w2_ref, b2_ref, o_ref):
    h = jnp.dot(x_ref[...], w1c_ref[...], preferred_element_type=jnp.float32)
    h = h + jnp.dot(nv_ref[...], w1n_ref[...], preferred_element_type=jnp.float32)
    h = jnp.maximum(h + b1_ref[...], 0.0)
    o = jnp.sum(h * w2_ref[...], axis=1, keepdims=True) + b2_ref[0, 0]
    o_ref[...] = jax.nn.sigmoid(o)


def _mlp_tc(cat_flat, num_values, w1c, w1n, b1f, w2r, b2s):
    blk = 1024
    return pl.pallas_call(
        _mlp_body,
        grid=(B // blk,),
        in_specs=[
            pl.BlockSpec((blk, F_CAT * D), lambda i: (i, 0)),
            pl.BlockSpec((blk, F_NUM), lambda i: (i, 0)),
            pl.BlockSpec((F_CAT * D, H), lambda i: (0, 0)),
            pl.BlockSpec((F_NUM, H), lambda i: (0, 0)),
            pl.BlockSpec((1, H), lambda i: (0, 0)),
            pl.BlockSpec((1, H), lambda i: (0, 0)),
            pl.BlockSpec((1, 1), lambda i: (0, 0)),
        ],
        out_specs=pl.BlockSpec((blk, 1), lambda i: (i, 0)),
        out_shape=jax.ShapeDtypeStruct((B, 1), jnp.float32),
    )(cat_flat, num_values, w1c, w1n, b1f, w2r, b2s)


def kernel(cat_indices, num_values, emb_tables, num_W, num_b, W1, b1, W2, b2):
    emb_t = emb_tables.transpose(0, 2, 1)
    tail_lin = emb_tables[:, 99968:, :].reshape(-1)
    table1d = _transpose_sc(emb_t, tail_lin)
    table = table1d.reshape(F_CAT * VP, D)
    offs = (jnp.arange(F_CAT, dtype=jnp.int32) * VP)[None, :]
    idx = (cat_indices.astype(jnp.int32) + offs).reshape(NW, NCH, CH)
    rows = _gather_sc(table, idx)
    cat_flat = rows.reshape(B, F_CAT * D)

    w1c = W1[: F_CAT * D]
    w1r = W1[F_CAT * D :]
    w1n = jnp.einsum("fd,fdh->fh", num_W, w1r.reshape(F_NUM, D, H))
    b1f = (b1 + num_b.reshape(1, F_NUM * D) @ w1r).reshape(1, H)
    w2r = W2.reshape(1, H)
    b2s = b2.reshape(1, 1)
    return _mlp_tc(cat_flat, num_values, w1c, w1n, b1f, w2r, b2s)
